# baseline (speedup 1.0000x reference)
import functools

import jax
import jax.numpy as jnp
from jax import lax
from jax.experimental import pallas as pl
from jax.experimental.pallas import tpu as pltpu
from jax.experimental.pallas import tpu_sc as plsc

_N_USERS = 50000
_N_ITEMS = 50000
_DIM = 32
_N_NODES = _N_USERS + _N_ITEMS
_N_EDGES = 1600000

_NC = 2
_NS = 16
_CH = 128
_GATHER_ON = True
_EDGE_ON = True
_SCALE_ON = True
_R = 3
_CPT = 783
_NCH = _CPT * _NS
_WORDS = 3 * _CH

_ROWS_PER_CORE = _N_NODES // _NC
_ACC_ROWS = 50048
_DUMP_ROW = _ROWS_PER_CORE
_ZROWS = _ACC_ROWS // _NS
_DRAIN = 3128
_DRAIN_LAST = _ROWS_PER_CORE - 15 * _DRAIN


_BCAST_DNUMS = lax.GatherDimensionNumbers(
    offset_dims=(), collapsed_slice_dims=(0,), start_index_map=(0,))


def _lane_bcast(v16, i):
  idx = jnp.full((16, 1), i, jnp.int32)
  return lax.gather(v16, idx, dimension_numbers=_BCAST_DNUMS,
                    slice_sizes=(1,),
                    mode=lax.GatherScatterMode.PROMISE_IN_BOUNDS)


def _scale_and_index(idxb, gath, scl, lidx, base_row):
  @plsc.parallel_loop(0, _CH // 16, unroll=4)
  def g_body(g):
    off = g * 16
    rows16 = idxb[pl.ds(off, 16)]
    local = rows16 - base_row
    ok = (local >= 0) & (local < _ROWS_PER_CORE)
    dump = _DUMP_ROW + (rows16 & 31)
    lidx[pl.ds(off, 16)] = jnp.where(ok, local, dump)
    v16 = plsc.bitcast(idxb[pl.ds(2 * _CH + off, 16)], jnp.float32)
    for i in range(16 if _SCALE_ON else 0):
      e = off + i
      m = _lane_bcast(v16, i)
      scl[e, pl.ds(0, 16)] = gath[e, pl.ds(0, 16)] * m
      scl[e, pl.ds(16, 16)] = gath[e, pl.ds(16, 16)] * m


def _layer_body(packed_h, emb_h, out_h, accum,
                ibuf, gath, scl, lidx, zbuf, isem, gsem, ssem):
  cid = lax.axis_index("c")
  sid = lax.axis_index("s")
  base_row = cid * _ROWS_PER_CORE

  @plsc.parallel_loop(0, _CH, unroll=4)
  def _zero_zbuf(r):
    z = jnp.zeros((16,), jnp.float32)
    zbuf[r, pl.ds(0, 16)] = z
    zbuf[r, pl.ds(16, 16)] = z

  zbase = sid * _ZROWS
  for q in range(_ZROWS // _CH):
    pltpu.sync_copy(zbuf, accum.at[pl.ds(zbase + q * _CH, _CH)])
  rem = _ZROWS % _CH
  if rem:
    pltpu.sync_copy(zbuf.at[pl.ds(0, rem)],
                    accum.at[pl.ds(zbase + _ZROWS - rem, rem)])
  plsc.subcore_barrier()

  c0 = sid * _CPT

  def issue_idx(chunk, s):
    pltpu.async_copy(packed_h.at[pl.ds(chunk * _WORDS, _WORDS)], ibuf[s],
                     isem[s])

  def wait_idx(s):
    pltpu.make_async_copy(packed_h.at[pl.ds(0, _WORDS)], ibuf[s],
                          isem[s]).wait()

  def issue_gather(s):
    pltpu.async_copy(emb_h.at[ibuf[s].at[pl.ds(_CH, _CH)]], gath[s], gsem[s])

  def wait_gather(s):
    pltpu.make_async_copy(emb_h.at[ibuf[s].at[pl.ds(_CH, _CH)]], gath[s],
                          gsem[s]).wait()

  def issue_scatter(s):
    pltpu.async_copy(scl[s], accum.at[lidx[s]], ssem[s], add=True)

  def wait_scatter(s):
    pltpu.make_async_copy(scl[s], accum.at[lidx[s]], ssem[s]).wait()

  for k in range(_R if _EDGE_ON else 0):
    issue_idx(c0 + k, k)
  for k in range(_R - 1 if _EDGE_ON else 0):
    wait_idx(k)
    if _GATHER_ON:
      issue_gather(k)

  def visit(c, s, first, last, tail_gather):
    if _GATHER_ON:
      wait_gather(s)
    if not first:
      wait_scatter(s)
    _scale_and_index(ibuf[s], gath[s], scl[s], lidx[s], base_row)
    if not last:
      issue_idx(c + _R, s)
    if (not last) or tail_gather:
      sp = (s + _R - 1) % _R
      wait_idx(sp)
      if _GATHER_ON:
        issue_gather(sp)
    issue_scatter(s)

  for s in range(_R if _EDGE_ON else 0):
    visit(c0 + s, s, True, False, False)

  def loop_body(j, carry):
    cb = c0 + _R * j
    for s in range(_R):
      visit(cb + s, s, False, False, False)
    return carry

  if _EDGE_ON:
    lax.fori_loop(1, _CPT // _R - 1, loop_body, 0)

  cl = c0 + _CPT - _R
  for s in range(_R if _EDGE_ON else 0):
    visit(cl + s, s, False, True, s == 0)

  for s in range(_R if _EDGE_ON else 0):
    wait_scatter(s)

  plsc.subcore_barrier()

  @pl.when(sid < _NS - 1)
  def _drain_main():
    pltpu.sync_copy(
        accum.at[pl.ds(sid * _DRAIN, _DRAIN)],
        out_h.at[pl.ds(cid * _ROWS_PER_CORE + sid * _DRAIN, _DRAIN)])

  @pl.when(sid == _NS - 1)
  def _drain_last():
    pltpu.sync_copy(
        accum.at[pl.ds((_NS - 1) * _DRAIN, _DRAIN_LAST)],
        out_h.at[pl.ds(cid * _ROWS_PER_CORE + (_NS - 1) * _DRAIN,
                       _DRAIN_LAST)])


_sc_layer = functools.partial(
    pl.kernel,
    out_type=jax.ShapeDtypeStruct((_N_NODES, _DIM), jnp.float32),
    mesh=plsc.VectorSubcoreMesh(
        core_axis_name="c", subcore_axis_name="s",
        num_cores=_NC, num_subcores=_NS),
    scratch_types=[
        pltpu.VMEM_SHARED((_ACC_ROWS, _DIM), jnp.float32),
        [pltpu.VMEM((_WORDS,), jnp.int32) for _ in range(_R)],
        [pltpu.VMEM((_CH, _DIM), jnp.float32) for _ in range(_R)],
        [pltpu.VMEM((_CH, _DIM), jnp.float32) for _ in range(_R)],
        [pltpu.VMEM((_CH,), jnp.int32) for _ in range(_R)],
        pltpu.VMEM((_CH, _DIM), jnp.float32),
        [pltpu.SemaphoreType.DMA for _ in range(_R)],
        [pltpu.SemaphoreType.DMA for _ in range(_R)],
        [pltpu.SemaphoreType.DMA for _ in range(_R)],
    ],
    compiler_params=pltpu.CompilerParams(
        needs_layout_passes=False, use_tc_tiling_on_sc=False),
)(_layer_body)


def _mean_body(a, b, c, d, o):
  o[...] = (a[...] + b[...] + c[...] + d[...]) * 0.25


_mean4 = pl.pallas_call(
    _mean_body,
    grid=(50,),
    in_specs=[pl.BlockSpec((_N_NODES // 50, _DIM), lambda i: (i, 0))] * 4,
    out_specs=pl.BlockSpec((_N_NODES // 50, _DIM), lambda i: (i, 0)),
    out_shape=jax.ShapeDtypeStruct((_N_NODES, _DIM), jnp.float32),
)


def _pack_edges(adj_indices, adj_values):
  pad = _NCH * _CH - _N_EDGES
  rows = jnp.concatenate([adj_indices[0], jnp.zeros((pad,), jnp.int32)])
  cols = jnp.concatenate([adj_indices[1], jnp.zeros((pad,), jnp.int32)])
  vals = jnp.concatenate([adj_values, jnp.zeros((pad,), jnp.float32)])
  vbits = lax.bitcast_convert_type(vals, jnp.int32)
  packed = jnp.stack(
      [rows.reshape(_NCH, _CH), cols.reshape(_NCH, _CH),
       vbits.reshape(_NCH, _CH)], axis=1)
  return packed.reshape(-1)


def kernel(adj_indices, adj_values, user_emb, item_emb):
  packed = _pack_edges(adj_indices, adj_values)
  emb0 = jnp.concatenate([user_emb, item_emb], axis=0)
  emb1 = _sc_layer(packed, emb0)
  emb2 = _sc_layer(packed, emb1)
  emb3 = _sc_layer(packed, emb2)
  out = _mean4(emb0, emb1, emb2, emb3)
  return (out[:_N_USERS], out[_N_USERS:])


# speedup vs baseline: 1.1761x; 1.1761x over previous
import functools

import jax
import jax.numpy as jnp
from jax import lax
from jax.experimental import pallas as pl
from jax.experimental.pallas import tpu as pltpu
from jax.experimental.pallas import tpu_sc as plsc

_N_USERS = 50000
_N_ITEMS = 50000
_DIM = 32
_N_NODES = _N_USERS + _N_ITEMS
_N_EDGES = 1600000

_NC = 2
_NS = 16
_CH = 128
_R = 3
_CPT = 783
_NCH = _CPT * _NS
_WORDS = 3 * _CH

_TCAP_CH = 786
_TCAP_E = _TCAP_CH * _CH

_ROWS_PER_CORE = _N_NODES // _NC
_ACC_ROWS = 50048
_ZROWS = _ACC_ROWS // _NS
_DRAIN = 3128
_DRAIN_LAST = _ROWS_PER_CORE - 15 * _DRAIN


_BCAST_DNUMS = lax.GatherDimensionNumbers(
    offset_dims=(), collapsed_slice_dims=(0,), start_index_map=(0,))


def _lane_bcast(v16, i):
  idx = jnp.full((16, 1), i, jnp.int32)
  return lax.gather(v16, idx, dimension_numbers=_BCAST_DNUMS,
                    slice_sizes=(1,),
                    mode=lax.GatherScatterMode.PROMISE_IN_BOUNDS)


def _part_body(packed_h, rows_h, cols_h, vals_h, cnt_h,
               pibuf, stg_r, stg_c, stg_v, cbuf, psem):
  cid = lax.axis_index("c")
  sid = lax.axis_index("s")
  base_row = cid * _ROWS_PER_CORE
  wid = cid * _NS + sid
  seg = wid * _TCAP_E
  c0 = sid * _CPT
  lane = lax.iota(jnp.int32, 16)
  zeros = jnp.zeros((16,), jnp.int32)

  def issue(c, p):
    pltpu.async_copy(packed_h.at[pl.ds((c0 + c) * _WORDS, _WORDS)], pibuf[p],
                     psem[p])

  def wait(p):
    pltpu.make_async_copy(packed_h.at[pl.ds(0, _WORDS)], pibuf[p],
                          psem[p]).wait()

  issue(0, 0)
  issue(1, 1)

  def proc_chunk(buf, soff):
    def g_body(g, so):
      off = g * 16
      r16 = buf[pl.ds(off, 16)]
      c16 = buf[pl.ds(_CH + off, 16)]
      v16 = buf[pl.ds(2 * _CH + off, 16)]
      local = r16 - base_row
      ok = (local >= 0) & (local < _ROWS_PER_CORE)
      oki = jnp.where(ok, 1, 0)
      cums = plsc.cumsum(oki)
      pos = so + cums - oki
      plsc.store_scatter(stg_r, [pos], local, mask=ok)
      plsc.store_scatter(stg_c, [pos], c16, mask=ok)
      plsc.store_scatter(stg_v, [pos], v16, mask=ok)
      return so + jnp.sum(oki)

    return lax.fori_loop(0, _CH // 16, g_body, soff)

  def flush512(soff, fe):
    do = soff >= 512

    @pl.when(do)
    def _():
      dst = seg + pl.multiple_of(fe, 512)
      pltpu.sync_copy(stg_r.at[pl.ds(0, 512)], rows_h.at[pl.ds(dst, 512)])
      pltpu.sync_copy(stg_c.at[pl.ds(0, 512)], cols_h.at[pl.ds(dst, 512)])
      pltpu.sync_copy(stg_v.at[pl.ds(0, 512)], vals_h.at[pl.ds(dst, 512)])
      for q in range(8):
        o = q * 16
        stg_r[pl.ds(o, 16)] = stg_r[pl.ds(512 + o, 16)]
        stg_c[pl.ds(o, 16)] = stg_c[pl.ds(512 + o, 16)]
        stg_v[pl.ds(o, 16)] = stg_v[pl.ds(512 + o, 16)]

    doi = do.astype(jnp.int32)
    return soff - 512 * doi, fe + 512 * doi

  def loop_body(j, carry):
    soff, fe = carry
    for p in range(2):
      c = 2 * j + p
      wait(p)
      soff = proc_chunk(pibuf[p], soff)

      @pl.when(c + 2 < _CPT)
      def _():
        issue(c + 2, p)

      soff, fe = flush512(soff, fe)
    return soff, fe

  soff, fe = lax.fori_loop(0, _CPT // 2, loop_body,
                           (jnp.int32(0), jnp.int32(0)))
  wait(0)
  soff = proc_chunk(pibuf[0], soff)
  soff, fe = flush512(soff, fe)

  def pad16(so):
    k = jnp.minimum((-so) & 127, 16)
    zmask = lane < k
    pos = so + lane
    plsc.store_scatter(stg_r, [pos], zeros, mask=zmask)
    plsc.store_scatter(stg_c, [pos], zeros, mask=zmask)
    plsc.store_scatter(stg_v, [pos], zeros, mask=zmask)
    return so + k

  soff = lax.while_loop(lambda so: (so & 127) != 0, pad16, soff)

  def drain_chunk(carry):
    sidx, fe2 = carry
    src = sidx * _CH
    dst = seg + pl.multiple_of(fe2, _CH)
    pltpu.sync_copy(stg_r.at[pl.ds(src, _CH)], rows_h.at[pl.ds(dst, _CH)])
    pltpu.sync_copy(stg_c.at[pl.ds(src, _CH)], cols_h.at[pl.ds(dst, _CH)])
    pltpu.sync_copy(stg_v.at[pl.ds(src, _CH)], vals_h.at[pl.ds(dst, _CH)])
    return sidx + 1, fe2 + _CH

  _, fe = lax.while_loop(lambda c: c[0] * _CH < soff, drain_chunk,
                         (jnp.int32(0), fe))

  for q in range(8):
    o = q * 16
    stg_r[pl.ds(o, 16)] = zeros
    stg_c[pl.ds(o, 16)] = zeros
    stg_v[pl.ds(o, 16)] = zeros

  nch = fe // _CH
  target = jnp.maximum(6, ((nch + _R - 1) // _R) * _R)

  def pad_chunk(n):
    dst = seg + pl.multiple_of(n * _CH, _CH)
    pltpu.sync_copy(stg_r.at[pl.ds(0, _CH)], rows_h.at[pl.ds(dst, _CH)])
    pltpu.sync_copy(stg_c.at[pl.ds(0, _CH)], cols_h.at[pl.ds(dst, _CH)])
    pltpu.sync_copy(stg_v.at[pl.ds(0, _CH)], vals_h.at[pl.ds(dst, _CH)])
    return n + 1

  nch = lax.while_loop(lambda n: n < target, pad_chunk, nch)

  cbuf[pl.ds(0, 16)] = jnp.full((16,), 1, jnp.int32) * nch
  pltpu.sync_copy(cbuf, cnt_h.at[pl.ds(wid * 16, 16)])


_sc_partition = functools.partial(
    pl.kernel,
    out_type=[
        jax.ShapeDtypeStruct((_NC * _NS * _TCAP_E,), jnp.int32),
        jax.ShapeDtypeStruct((_NC * _NS * _TCAP_E,), jnp.int32),
        jax.ShapeDtypeStruct((_NC * _NS * _TCAP_E,), jnp.int32),
        jax.ShapeDtypeStruct((_NC * _NS * 16,), jnp.int32),
    ],
    mesh=plsc.VectorSubcoreMesh(
        core_axis_name="c", subcore_axis_name="s",
        num_cores=_NC, num_subcores=_NS),
    scratch_types=[
        [pltpu.VMEM((_WORDS,), jnp.int32) for _ in range(2)],
        pltpu.VMEM((640,), jnp.int32),
        pltpu.VMEM((640,), jnp.int32),
        pltpu.VMEM((640,), jnp.int32),
        pltpu.VMEM((16,), jnp.int32),
        [pltpu.SemaphoreType.DMA for _ in range(2)],
    ],
    compiler_params=pltpu.CompilerParams(
        needs_layout_passes=False, use_tc_tiling_on_sc=False),
)(_part_body)


def _scale_and_index(idxb, gath, scl, lidx):
  @plsc.parallel_loop(0, _CH // 16, unroll=4)
  def g_body(g):
    off = g * 16
    lidx[pl.ds(off, 16)] = idxb[pl.ds(off, 16)]
    v16 = plsc.bitcast(idxb[pl.ds(2 * _CH + off, 16)], jnp.float32)
    for i in range(16):
      e = off + i
      m = _lane_bcast(v16, i)
      scl[e, pl.ds(0, 16)] = gath[e, pl.ds(0, 16)] * m
      scl[e, pl.ds(16, 16)] = gath[e, pl.ds(16, 16)] * m


def _layer_body(rows_h, cols_h, vals_h, cnt_h, emb_h, out_h, accum,
                ibuf, gath, scl, lidx, zbuf, cntb, isem, gsem, ssem):
  cid = lax.axis_index("c")
  sid = lax.axis_index("s")
  wid = cid * _NS + sid
  seg = wid * _TCAP_E
  lane = lax.iota(jnp.int32, 16)

  @plsc.parallel_loop(0, _CH, unroll=4)
  def _zero_zbuf(r):
    z = jnp.zeros((16,), jnp.float32)
    zbuf[r, pl.ds(0, 16)] = z
    zbuf[r, pl.ds(16, 16)] = z

  zbase = sid * _ZROWS
  for q in range(_ZROWS // _CH):
    pltpu.sync_copy(zbuf, accum.at[pl.ds(zbase + q * _CH, _CH)])
  rem = _ZROWS % _CH
  if rem:
    pltpu.sync_copy(zbuf.at[pl.ds(0, rem)],
                    accum.at[pl.ds(zbase + _ZROWS - rem, rem)])

  pltpu.sync_copy(cnt_h.at[pl.ds(wid * 16, 16)], cntb)
  cnt16 = cntb[pl.ds(0, 16)]
  nch = jnp.sum(jnp.where(lane == 0, cnt16, 0))
  nrounds = nch // _R

  plsc.subcore_barrier()

  def issue_idx(chunk, s):
    base = seg + chunk * _CH
    pltpu.async_copy(rows_h.at[pl.ds(base, _CH)], ibuf[s].at[pl.ds(0, _CH)],
                     isem[s])
    pltpu.async_copy(cols_h.at[pl.ds(base, _CH)], ibuf[s].at[pl.ds(_CH, _CH)],
                     isem[s])
    pltpu.async_copy(vals_h.at[pl.ds(base, _CH)],
                     ibuf[s].at[pl.ds(2 * _CH, _CH)], isem[s])

  def wait_idx(s):
    for f in range(3):
      pltpu.make_async_copy(rows_h.at[pl.ds(0, _CH)],
                            ibuf[s].at[pl.ds(f * _CH, _CH)], isem[s]).wait()

  def issue_gather(s):
    pltpu.async_copy(emb_h.at[ibuf[s].at[pl.ds(_CH, _CH)]], gath[s], gsem[s])

  def wait_gather(s):
    pltpu.make_async_copy(emb_h.at[ibuf[s].at[pl.ds(_CH, _CH)]], gath[s],
                          gsem[s]).wait()

  def issue_scatter(s):
    pltpu.async_copy(scl[s], accum.at[lidx[s]], ssem[s], add=True)

  def wait_scatter(s):
    pltpu.make_async_copy(scl[s], accum.at[lidx[s]], ssem[s]).wait()

  for k in range(_R):
    issue_idx(k, k)
  for k in range(_R - 1):
    wait_idx(k)
    issue_gather(k)

  def visit(c, s, first, last, tail_gather):
    wait_gather(s)
    if not first:
      wait_scatter(s)
    _scale_and_index(ibuf[s], gath[s], scl[s], lidx[s])
    if not last:
      issue_idx(c + _R, s)
    if (not last) or tail_gather:
      sp = (s + _R - 1) % _R
      wait_idx(sp)
      issue_gather(sp)
    issue_scatter(s)

  for s in range(_R):
    visit(s, s, True, False, False)

  def loop_body(j, carry):
    cb = _R * j
    for s in range(_R):
      visit(cb + s, s, False, False, False)
    return carry

  lax.fori_loop(1, nrounds - 1, loop_body, 0)

  cl = (nrounds - 1) * _R
  for s in range(_R):
    visit(cl + s, s, False, True, s == 0)

  for s in range(_R):
    wait_scatter(s)

  plsc.subcore_barrier()

  @pl.when(sid < _NS - 1)
  def _drain_main():
    pltpu.sync_copy(
        accum.at[pl.ds(sid * _DRAIN, _DRAIN)],
        out_h.at[pl.ds(cid * _ROWS_PER_CORE + sid * _DRAIN, _DRAIN)])

  @pl.when(sid == _NS - 1)
  def _drain_last():
    pltpu.sync_copy(
        accum.at[pl.ds((_NS - 1) * _DRAIN, _DRAIN_LAST)],
        out_h.at[pl.ds(cid * _ROWS_PER_CORE + (_NS - 1) * _DRAIN,
                       _DRAIN_LAST)])


_sc_layer = functools.partial(
    pl.kernel,
    out_type=jax.ShapeDtypeStruct((_N_NODES, _DIM), jnp.float32),
    mesh=plsc.VectorSubcoreMesh(
        core_axis_name="c", subcore_axis_name="s",
        num_cores=_NC, num_subcores=_NS),
    scratch_types=[
        pltpu.VMEM_SHARED((_ACC_ROWS, _DIM), jnp.float32),
        [pltpu.VMEM((_WORDS,), jnp.int32) for _ in range(_R)],
        [pltpu.VMEM((_CH, _DIM), jnp.float32) for _ in range(_R)],
        [pltpu.VMEM((_CH, _DIM), jnp.float32) for _ in range(_R)],
        [pltpu.VMEM((_CH,), jnp.int32) for _ in range(_R)],
        pltpu.VMEM((_CH, _DIM), jnp.float32),
        pltpu.VMEM((16,), jnp.int32),
        [pltpu.SemaphoreType.DMA for _ in range(_R)],
        [pltpu.SemaphoreType.DMA for _ in range(_R)],
        [pltpu.SemaphoreType.DMA for _ in range(_R)],
    ],
    compiler_params=pltpu.CompilerParams(
        needs_layout_passes=False, use_tc_tiling_on_sc=False),
)(_layer_body)


def _mean_body(a, b, c, d, o):
  o[...] = (a[...] + b[...] + c[...] + d[...]) * 0.25


_mean4 = pl.pallas_call(
    _mean_body,
    grid=(50,),
    in_specs=[pl.BlockSpec((_N_NODES // 50, _DIM), lambda i: (i, 0))] * 4,
    out_specs=pl.BlockSpec((_N_NODES // 50, _DIM), lambda i: (i, 0)),
    out_shape=jax.ShapeDtypeStruct((_N_NODES, _DIM), jnp.float32),
)


def _pack_edges(adj_indices, adj_values):
  pad = _NCH * _CH - _N_EDGES
  rows = jnp.concatenate([adj_indices[0], jnp.zeros((pad,), jnp.int32)])
  cols = jnp.concatenate([adj_indices[1], jnp.zeros((pad,), jnp.int32)])
  vals = jnp.concatenate([adj_values, jnp.zeros((pad,), jnp.float32)])
  vbits = lax.bitcast_convert_type(vals, jnp.int32)
  packed = jnp.stack(
      [rows.reshape(_NCH, _CH), cols.reshape(_NCH, _CH),
       vbits.reshape(_NCH, _CH)], axis=1)
  return packed.reshape(-1)


def kernel(adj_indices, adj_values, user_emb, item_emb):
  packed = _pack_edges(adj_indices, adj_values)
  rows_l, cols_l, vals_l, cnts = _sc_partition(packed)
  emb0 = jnp.concatenate([user_emb, item_emb], axis=0)
  emb1 = _sc_layer(rows_l, cols_l, vals_l, cnts, emb0)
  emb2 = _sc_layer(rows_l, cols_l, vals_l, cnts, emb1)
  emb3 = _sc_layer(rows_l, cols_l, vals_l, cnts, emb2)
  out = _mean4(emb0, emb1, emb2, emb3)
  return (out[:_N_USERS], out[_N_USERS:])


# speedup vs baseline: 1.2013x; 1.0214x over previous
import functools

import jax
import jax.numpy as jnp
from jax import lax
from jax.experimental import pallas as pl
from jax.experimental.pallas import tpu as pltpu
from jax.experimental.pallas import tpu_sc as plsc

_N_USERS = 50000
_N_ITEMS = 50000
_DIM = 32
_N_NODES = _N_USERS + _N_ITEMS
_N_EDGES = 1600000

_NC = 2
_NS = 16
_CH = 128
_R = 3
_CPT = 783
_NCH = _CPT * _NS
_WORDS = 3 * _CH

_TCAP_CH = 786
_TCAP_E = _TCAP_CH * _CH

_ROWS_PER_CORE = _N_NODES // _NC
_ACC_ROWS = 50048
_ZROWS = _ACC_ROWS // _NS
_DRAIN = 3128
_DRAIN_LAST = _ROWS_PER_CORE - 15 * _DRAIN


_BCAST_DNUMS = lax.GatherDimensionNumbers(
    offset_dims=(), collapsed_slice_dims=(0,), start_index_map=(0,))


def _lane_bcast(v16, i):
  idx = jnp.full((16, 1), i, jnp.int32)
  return lax.gather(v16, idx, dimension_numbers=_BCAST_DNUMS,
                    slice_sizes=(1,),
                    mode=lax.GatherScatterMode.PROMISE_IN_BOUNDS)


def _part_body(packed_h, rows_h, cols_h, vals_h, cnt_h,
               pibuf, stg_r, stg_c, stg_v, cbuf, psem):
  cid = lax.axis_index("c")
  sid = lax.axis_index("s")
  base_row = cid * _ROWS_PER_CORE
  wid = cid * _NS + sid
  seg = wid * _TCAP_E
  c0 = sid * _CPT
  lane = lax.iota(jnp.int32, 16)
  zeros = jnp.zeros((16,), jnp.int32)

  def issue(c, p):
    pltpu.async_copy(packed_h.at[pl.ds((c0 + c) * _WORDS, _WORDS)], pibuf[p],
                     psem[p])

  def wait(p):
    pltpu.make_async_copy(packed_h.at[pl.ds(0, _WORDS)], pibuf[p],
                          psem[p]).wait()

  issue(0, 0)
  issue(1, 1)

  def proc_chunk(buf, soff):
    def g_body(g, so):
      off = g * 16
      r16 = buf[pl.ds(off, 16)]
      c16 = buf[pl.ds(_CH + off, 16)]
      v16 = buf[pl.ds(2 * _CH + off, 16)]
      local = r16 - base_row
      ok = (local >= 0) & (local < _ROWS_PER_CORE)
      oki = jnp.where(ok, 1, 0)
      cums = plsc.cumsum(oki)
      pos = so + cums - oki
      plsc.store_scatter(stg_r, [pos], local, mask=ok)
      plsc.store_scatter(stg_c, [pos], c16, mask=ok)
      plsc.store_scatter(stg_v, [pos], v16, mask=ok)
      return so + jnp.sum(oki)

    return lax.fori_loop(0, _CH // 16, g_body, soff)

  def flush512(soff, fe):
    do = soff >= 512

    @pl.when(do)
    def _():
      dst = seg + pl.multiple_of(fe, 512)
      pltpu.sync_copy(stg_r.at[pl.ds(0, 512)], rows_h.at[pl.ds(dst, 512)])
      pltpu.sync_copy(stg_c.at[pl.ds(0, 512)], cols_h.at[pl.ds(dst, 512)])
      pltpu.sync_copy(stg_v.at[pl.ds(0, 512)], vals_h.at[pl.ds(dst, 512)])
      for q in range(8):
        o = q * 16
        stg_r[pl.ds(o, 16)] = stg_r[pl.ds(512 + o, 16)]
        stg_c[pl.ds(o, 16)] = stg_c[pl.ds(512 + o, 16)]
        stg_v[pl.ds(o, 16)] = stg_v[pl.ds(512 + o, 16)]

    doi = do.astype(jnp.int32)
    return soff - 512 * doi, fe + 512 * doi

  def loop_body(j, carry):
    soff, fe = carry
    for p in range(2):
      c = 2 * j + p
      wait(p)
      soff = proc_chunk(pibuf[p], soff)

      @pl.when(c + 2 < _CPT)
      def _():
        issue(c + 2, p)

      soff, fe = flush512(soff, fe)
    return soff, fe

  soff, fe = lax.fori_loop(0, _CPT // 2, loop_body,
                           (jnp.int32(0), jnp.int32(0)))
  wait(0)
  soff = proc_chunk(pibuf[0], soff)
  soff, fe = flush512(soff, fe)

  def pad16(so):
    k = jnp.minimum((-so) & 127, 16)
    zmask = lane < k
    pos = so + lane
    plsc.store_scatter(stg_r, [pos], zeros, mask=zmask)
    plsc.store_scatter(stg_c, [pos], zeros, mask=zmask)
    plsc.store_scatter(stg_v, [pos], zeros, mask=zmask)
    return so + k

  soff = lax.while_loop(lambda so: (so & 127) != 0, pad16, soff)

  def drain_chunk(carry):
    sidx, fe2 = carry
    src = sidx * _CH
    dst = seg + pl.multiple_of(fe2, _CH)
    pltpu.sync_copy(stg_r.at[pl.ds(src, _CH)], rows_h.at[pl.ds(dst, _CH)])
    pltpu.sync_copy(stg_c.at[pl.ds(src, _CH)], cols_h.at[pl.ds(dst, _CH)])
    pltpu.sync_copy(stg_v.at[pl.ds(src, _CH)], vals_h.at[pl.ds(dst, _CH)])
    return sidx + 1, fe2 + _CH

  _, fe = lax.while_loop(lambda c: c[0] * _CH < soff, drain_chunk,
                         (jnp.int32(0), fe))

  for q in range(8):
    o = q * 16
    stg_r[pl.ds(o, 16)] = zeros
    stg_c[pl.ds(o, 16)] = zeros
    stg_v[pl.ds(o, 16)] = zeros

  nch = fe // _CH
  target = jnp.maximum(6, ((nch + _R - 1) // _R) * _R)

  def pad_chunk(n):
    dst = seg + pl.multiple_of(n * _CH, _CH)
    pltpu.sync_copy(stg_r.at[pl.ds(0, _CH)], rows_h.at[pl.ds(dst, _CH)])
    pltpu.sync_copy(stg_c.at[pl.ds(0, _CH)], cols_h.at[pl.ds(dst, _CH)])
    pltpu.sync_copy(stg_v.at[pl.ds(0, _CH)], vals_h.at[pl.ds(dst, _CH)])
    return n + 1

  nch = lax.while_loop(lambda n: n < target, pad_chunk, nch)

  cbuf[pl.ds(0, 16)] = jnp.full((16,), 1, jnp.int32) * nch
  pltpu.sync_copy(cbuf, cnt_h.at[pl.ds(wid * 16, 16)])


_sc_partition = functools.partial(
    pl.kernel,
    out_type=[
        jax.ShapeDtypeStruct((_NC * _NS * _TCAP_E,), jnp.int32),
        jax.ShapeDtypeStruct((_NC * _NS * _TCAP_E,), jnp.int32),
        jax.ShapeDtypeStruct((_NC * _NS * _TCAP_E,), jnp.int32),
        jax.ShapeDtypeStruct((_NC * _NS * 16,), jnp.int32),
    ],
    mesh=plsc.VectorSubcoreMesh(
        core_axis_name="c", subcore_axis_name="s",
        num_cores=_NC, num_subcores=_NS),
    scratch_types=[
        [pltpu.VMEM((_WORDS,), jnp.int32) for _ in range(2)],
        pltpu.VMEM((640,), jnp.int32),
        pltpu.VMEM((640,), jnp.int32),
        pltpu.VMEM((640,), jnp.int32),
        pltpu.VMEM((16,), jnp.int32),
        [pltpu.SemaphoreType.DMA for _ in range(2)],
    ],
    compiler_params=pltpu.CompilerParams(
        needs_layout_passes=False, use_tc_tiling_on_sc=False),
)(_part_body)


def _scale_and_index(idxb, gath, scl, lidx):
  @plsc.parallel_loop(0, _CH // 16, unroll=4)
  def g_body(g):
    off = g * 16
    lidx[pl.ds(off, 16)] = idxb[pl.ds(off, 16)]
    v16 = plsc.bitcast(idxb[pl.ds(2 * _CH + off, 16)], jnp.float32)
    for i in range(16):
      e = off + i
      m = _lane_bcast(v16, i)
      scl[e, pl.ds(0, 16)] = gath[e, pl.ds(0, 16)] * m
      scl[e, pl.ds(16, 16)] = gath[e, pl.ds(16, 16)] * m


def _layer_body(rows_h, cols_h, vals_h, cnt_h, emb_h, out_h, accum,
                ibuf, gath, scl, lidx, zbuf, cntb, isem, gsem, ssem):
  cid = lax.axis_index("c")
  sid = lax.axis_index("s")
  wid = cid * _NS + sid
  seg = wid * _TCAP_E
  lane = lax.iota(jnp.int32, 16)

  @plsc.parallel_loop(0, _CH, unroll=4)
  def _zero_zbuf(r):
    z = jnp.zeros((16,), jnp.float32)
    zbuf[r, pl.ds(0, 16)] = z
    zbuf[r, pl.ds(16, 16)] = z

  zbase = sid * _ZROWS
  for q in range(_ZROWS // _CH):
    pltpu.sync_copy(zbuf, accum.at[pl.ds(zbase + q * _CH, _CH)])
  rem = _ZROWS % _CH
  if rem:
    pltpu.sync_copy(zbuf.at[pl.ds(0, rem)],
                    accum.at[pl.ds(zbase + _ZROWS - rem, rem)])

  pltpu.sync_copy(cnt_h.at[pl.ds(wid * 16, 16)], cntb)
  cnt16 = cntb[pl.ds(0, 16)]
  nch = jnp.sum(jnp.where(lane == 0, cnt16, 0))
  nrounds = nch // _R

  plsc.subcore_barrier()

  def issue_idx(chunk, s):
    base = seg + chunk * _CH
    pltpu.async_copy(rows_h.at[pl.ds(base, _CH)], ibuf[s].at[pl.ds(0, _CH)],
                     isem[s])
    pltpu.async_copy(cols_h.at[pl.ds(base, _CH)], ibuf[s].at[pl.ds(_CH, _CH)],
                     isem[s])
    pltpu.async_copy(vals_h.at[pl.ds(base, _CH)],
                     ibuf[s].at[pl.ds(2 * _CH, _CH)], isem[s])

  def wait_idx(s):
    for f in range(3):
      pltpu.make_async_copy(rows_h.at[pl.ds(0, _CH)],
                            ibuf[s].at[pl.ds(f * _CH, _CH)], isem[s]).wait()

  def issue_gather(s):
    pltpu.async_copy(emb_h.at[ibuf[s].at[pl.ds(_CH, _CH)]], gath[s], gsem[s])

  def wait_gather(s):
    pltpu.make_async_copy(emb_h.at[ibuf[s].at[pl.ds(_CH, _CH)]], gath[s],
                          gsem[s]).wait()

  def issue_scatter(s):
    pltpu.async_copy(scl[s], accum.at[lidx[s]], ssem[s], add=True)

  def wait_scatter(s):
    pltpu.make_async_copy(scl[s], accum.at[lidx[s]], ssem[s]).wait()

  for k in range(_R):
    issue_idx(k, k)
  for k in range(_R - 1):
    wait_idx(k)
    issue_gather(k)

  def visit(c, s, first, last, tail_gather):
    wait_gather(s)
    if not first:
      wait_scatter(s)
    _scale_and_index(ibuf[s], gath[s], scl[s], lidx[s])
    if not last:
      issue_idx(c + _R, s)
    if (not last) or tail_gather:
      sp = (s + _R - 1) % _R
      wait_idx(sp)
      issue_gather(sp)
    issue_scatter(s)

  for s in range(_R):
    visit(s, s, True, False, False)

  def loop_body(j, carry):
    cb = _R * j
    for s in range(_R):
      visit(cb + s, s, False, False, False)
    return carry

  lax.fori_loop(1, nrounds - 1, loop_body, 0)

  cl = (nrounds - 1) * _R
  for s in range(_R):
    visit(cl + s, s, False, True, s == 0)

  for s in range(_R):
    wait_scatter(s)

  plsc.subcore_barrier()

  @pl.when(sid < _NS - 1)
  def _drain_main():
    pltpu.sync_copy(
        accum.at[pl.ds(sid * _DRAIN, _DRAIN)],
        out_h.at[pl.ds(cid * _ROWS_PER_CORE + sid * _DRAIN, _DRAIN)])

  @pl.when(sid == _NS - 1)
  def _drain_last():
    pltpu.sync_copy(
        accum.at[pl.ds((_NS - 1) * _DRAIN, _DRAIN_LAST)],
        out_h.at[pl.ds(cid * _ROWS_PER_CORE + (_NS - 1) * _DRAIN,
                       _DRAIN_LAST)])


_sc_layer = functools.partial(
    pl.kernel,
    out_type=jax.ShapeDtypeStruct((_N_NODES, _DIM), jnp.float32),
    mesh=plsc.VectorSubcoreMesh(
        core_axis_name="c", subcore_axis_name="s",
        num_cores=_NC, num_subcores=_NS),
    scratch_types=[
        pltpu.VMEM_SHARED((_ACC_ROWS, _DIM), jnp.float32),
        [pltpu.VMEM((_WORDS,), jnp.int32) for _ in range(_R)],
        [pltpu.VMEM((_CH, _DIM), jnp.float32) for _ in range(_R)],
        [pltpu.VMEM((_CH, _DIM), jnp.float32) for _ in range(_R)],
        [pltpu.VMEM((_CH,), jnp.int32) for _ in range(_R)],
        pltpu.VMEM((_CH, _DIM), jnp.float32),
        pltpu.VMEM((16,), jnp.int32),
        [pltpu.SemaphoreType.DMA for _ in range(_R)],
        [pltpu.SemaphoreType.DMA for _ in range(_R)],
        [pltpu.SemaphoreType.DMA for _ in range(_R)],
    ],
    compiler_params=pltpu.CompilerParams(
        needs_layout_passes=False, use_tc_tiling_on_sc=False),
)(_layer_body)


def _mean_body(au, ai, bu, bi, cu, ci, du, di, ou, oi):
  ou[...] = (au[...] + bu[...] + cu[...] + du[...]) * 0.25
  oi[...] = (ai[...] + bi[...] + ci[...] + di[...]) * 0.25


_MB = 25
_MROWS = _N_USERS // _MB
_spec_u = pl.BlockSpec((_MROWS, _DIM), lambda i: (i, 0))
_spec_i = pl.BlockSpec((_MROWS, _DIM), lambda i: (i + _MB, 0))

_mean4 = pl.pallas_call(
    _mean_body,
    grid=(_MB,),
    in_specs=[_spec_u, _spec_i] * 4,
    out_specs=[pl.BlockSpec((_MROWS, _DIM), lambda i: (i, 0))] * 2,
    out_shape=[jax.ShapeDtypeStruct((_N_USERS, _DIM), jnp.float32),
               jax.ShapeDtypeStruct((_N_ITEMS, _DIM), jnp.float32)],
)


def _pack_edges(adj_indices, adj_values):
  pad = _NCH * _CH - _N_EDGES
  rows = jnp.concatenate([adj_indices[0], jnp.zeros((pad,), jnp.int32)])
  cols = jnp.concatenate([adj_indices[1], jnp.zeros((pad,), jnp.int32)])
  vals = jnp.concatenate([adj_values, jnp.zeros((pad,), jnp.float32)])
  vbits = lax.bitcast_convert_type(vals, jnp.int32)
  packed = jnp.stack(
      [rows.reshape(_NCH, _CH), cols.reshape(_NCH, _CH),
       vbits.reshape(_NCH, _CH)], axis=1)
  return packed.reshape(-1)


def kernel(adj_indices, adj_values, user_emb, item_emb):
  packed = _pack_edges(adj_indices, adj_values)
  rows_l, cols_l, vals_l, cnts = _sc_partition(packed)
  emb0 = jnp.concatenate([user_emb, item_emb], axis=0)
  emb1 = _sc_layer(rows_l, cols_l, vals_l, cnts, emb0)
  emb2 = _sc_layer(rows_l, cols_l, vals_l, cnts, emb1)
  emb3 = _sc_layer(rows_l, cols_l, vals_l, cnts, emb2)
  user, item = _mean4(emb0, emb0, emb1, emb1, emb2, emb2, emb3, emb3)
  return (user, item)


# speedup vs baseline: 1.2176x; 1.0135x over previous
import functools

import jax
import jax.numpy as jnp
from jax import lax
from jax.experimental import pallas as pl
from jax.experimental.pallas import tpu as pltpu
from jax.experimental.pallas import tpu_sc as plsc

_N_USERS = 50000
_N_ITEMS = 50000
_DIM = 32
_N_NODES = _N_USERS + _N_ITEMS
_N_EDGES = 1600000

_NC = 2
_NS = 16
_CH = 128
_R = 3
_CPT = 783
_NCH = _CPT * _NS
_WORDS = 3 * _CH

_TCAP_CH = 786
_TCAP_E = _TCAP_CH * _CH

_ROWS_PER_CORE = _N_NODES // _NC
_ACC_ROWS = 50048
_ZROWS = _ACC_ROWS // _NS
_DRAIN = 3128
_DRAIN_LAST = _ROWS_PER_CORE - 15 * _DRAIN


_BCAST_DNUMS = lax.GatherDimensionNumbers(
    offset_dims=(), collapsed_slice_dims=(0,), start_index_map=(0,))


def _lane_bcast(v16, i):
  idx = jnp.full((16, 1), i, jnp.int32)
  return lax.gather(v16, idx, dimension_numbers=_BCAST_DNUMS,
                    slice_sizes=(1,),
                    mode=lax.GatherScatterMode.PROMISE_IN_BOUNDS)


def _part_body(packed_h, rows_h, cols_h, vals_h, cnt_h,
               pibuf, stg_r, stg_c, stg_v, cbuf, psem):
  cid = lax.axis_index("c")
  sid = lax.axis_index("s")
  base_row = cid * _ROWS_PER_CORE
  wid = cid * _NS + sid
  seg = wid * _TCAP_E
  c0 = sid * _CPT
  lane = lax.iota(jnp.int32, 16)
  zeros = jnp.zeros((16,), jnp.int32)

  def issue(c, p):
    pltpu.async_copy(packed_h.at[pl.ds((c0 + c) * _WORDS, _WORDS)], pibuf[p],
                     psem[p])

  def wait(p):
    pltpu.make_async_copy(packed_h.at[pl.ds(0, _WORDS)], pibuf[p],
                          psem[p]).wait()

  issue(0, 0)
  issue(1, 1)

  def proc_chunk(buf, soff):
    @plsc.parallel_loop(0, _CH // 16, unroll=4, carry=soff)
    def g_body(g, so):
      off = g * 16
      r16 = buf[pl.ds(off, 16)]
      c16 = buf[pl.ds(_CH + off, 16)]
      v16 = buf[pl.ds(2 * _CH + off, 16)]
      local = r16 - base_row
      ok = (local >= 0) & (local < _ROWS_PER_CORE)
      oki = jnp.where(ok, 1, 0)
      cums = plsc.cumsum(oki)
      pos = so + cums - oki
      plsc.store_scatter(stg_r, [pos], local, mask=ok)
      plsc.store_scatter(stg_c, [pos], c16, mask=ok)
      plsc.store_scatter(stg_v, [pos], v16, mask=ok)
      return so + jnp.sum(oki)

    return g_body

  def flush512(soff, fe):
    do = soff >= 512

    @pl.when(do)
    def _():
      dst = seg + pl.multiple_of(fe, 512)
      pltpu.sync_copy(stg_r.at[pl.ds(0, 512)], rows_h.at[pl.ds(dst, 512)])
      pltpu.sync_copy(stg_c.at[pl.ds(0, 512)], cols_h.at[pl.ds(dst, 512)])
      pltpu.sync_copy(stg_v.at[pl.ds(0, 512)], vals_h.at[pl.ds(dst, 512)])
      for q in range(8):
        o = q * 16
        stg_r[pl.ds(o, 16)] = stg_r[pl.ds(512 + o, 16)]
        stg_c[pl.ds(o, 16)] = stg_c[pl.ds(512 + o, 16)]
        stg_v[pl.ds(o, 16)] = stg_v[pl.ds(512 + o, 16)]

    doi = do.astype(jnp.int32)
    return soff - 512 * doi, fe + 512 * doi

  def loop_body(j, carry):
    soff, fe = carry
    for p in range(2):
      c = 2 * j + p
      wait(p)
      soff = proc_chunk(pibuf[p], soff)

      @pl.when(c + 2 < _CPT)
      def _():
        issue(c + 2, p)

      soff, fe = flush512(soff, fe)
    return soff, fe

  soff, fe = lax.fori_loop(0, _CPT // 2, loop_body,
                           (jnp.int32(0), jnp.int32(0)))
  wait(0)
  soff = proc_chunk(pibuf[0], soff)
  soff, fe = flush512(soff, fe)

  def pad16(so):
    k = jnp.minimum((-so) & 127, 16)
    zmask = lane < k
    pos = so + lane
    plsc.store_scatter(stg_r, [pos], zeros, mask=zmask)
    plsc.store_scatter(stg_c, [pos], zeros, mask=zmask)
    plsc.store_scatter(stg_v, [pos], zeros, mask=zmask)
    return so + k

  soff = lax.while_loop(lambda so: (so & 127) != 0, pad16, soff)

  def drain_chunk(carry):
    sidx, fe2 = carry
    src = sidx * _CH
    dst = seg + pl.multiple_of(fe2, _CH)
    pltpu.sync_copy(stg_r.at[pl.ds(src, _CH)], rows_h.at[pl.ds(dst, _CH)])
    pltpu.sync_copy(stg_c.at[pl.ds(src, _CH)], cols_h.at[pl.ds(dst, _CH)])
    pltpu.sync_copy(stg_v.at[pl.ds(src, _CH)], vals_h.at[pl.ds(dst, _CH)])
    return sidx + 1, fe2 + _CH

  _, fe = lax.while_loop(lambda c: c[0] * _CH < soff, drain_chunk,
                         (jnp.int32(0), fe))

  for q in range(8):
    o = q * 16
    stg_r[pl.ds(o, 16)] = zeros
    stg_c[pl.ds(o, 16)] = zeros
    stg_v[pl.ds(o, 16)] = zeros

  nch = fe // _CH
  target = jnp.maximum(6, ((nch + _R - 1) // _R) * _R)

  def pad_chunk(n):
    dst = seg + pl.multiple_of(n * _CH, _CH)
    pltpu.sync_copy(stg_r.at[pl.ds(0, _CH)], rows_h.at[pl.ds(dst, _CH)])
    pltpu.sync_copy(stg_c.at[pl.ds(0, _CH)], cols_h.at[pl.ds(dst, _CH)])
    pltpu.sync_copy(stg_v.at[pl.ds(0, _CH)], vals_h.at[pl.ds(dst, _CH)])
    return n + 1

  nch = lax.while_loop(lambda n: n < target, pad_chunk, nch)

  cbuf[pl.ds(0, 16)] = jnp.full((16,), 1, jnp.int32) * nch
  pltpu.sync_copy(cbuf, cnt_h.at[pl.ds(wid * 16, 16)])


_sc_partition = functools.partial(
    pl.kernel,
    out_type=[
        jax.ShapeDtypeStruct((_NC * _NS * _TCAP_E,), jnp.int32),
        jax.ShapeDtypeStruct((_NC * _NS * _TCAP_E,), jnp.int32),
        jax.ShapeDtypeStruct((_NC * _NS * _TCAP_E,), jnp.int32),
        jax.ShapeDtypeStruct((_NC * _NS * 16,), jnp.int32),
    ],
    mesh=plsc.VectorSubcoreMesh(
        core_axis_name="c", subcore_axis_name="s",
        num_cores=_NC, num_subcores=_NS),
    scratch_types=[
        [pltpu.VMEM((_WORDS,), jnp.int32) for _ in range(2)],
        pltpu.VMEM((640,), jnp.int32),
        pltpu.VMEM((640,), jnp.int32),
        pltpu.VMEM((640,), jnp.int32),
        pltpu.VMEM((16,), jnp.int32),
        [pltpu.SemaphoreType.DMA for _ in range(2)],
    ],
    compiler_params=pltpu.CompilerParams(
        needs_layout_passes=False, use_tc_tiling_on_sc=False),
)(_part_body)


def _scale_and_index(idxb, gath, scl, lidx):
  @plsc.parallel_loop(0, _CH // 16, unroll=4)
  def g_body(g):
    off = g * 16
    lidx[pl.ds(off, 16)] = idxb[pl.ds(off, 16)]
    v16 = plsc.bitcast(idxb[pl.ds(2 * _CH + off, 16)], jnp.float32)
    for i in range(16):
      e = off + i
      m = _lane_bcast(v16, i)
      scl[e, pl.ds(0, 16)] = gath[e, pl.ds(0, 16)] * m
      scl[e, pl.ds(16, 16)] = gath[e, pl.ds(16, 16)] * m


def _layer_body(rows_h, cols_h, vals_h, cnt_h, emb_h, out_h, accum,
                ibuf, gath, scl, lidx, zbuf, cntb, isem, gsem, ssem):
  cid = lax.axis_index("c")
  sid = lax.axis_index("s")
  wid = cid * _NS + sid
  seg = wid * _TCAP_E
  lane = lax.iota(jnp.int32, 16)

  @plsc.parallel_loop(0, _CH, unroll=4)
  def _zero_zbuf(r):
    z = jnp.zeros((16,), jnp.float32)
    zbuf[r, pl.ds(0, 16)] = z
    zbuf[r, pl.ds(16, 16)] = z

  zbase = sid * _ZROWS
  for q in range(_ZROWS // _CH):
    pltpu.sync_copy(zbuf, accum.at[pl.ds(zbase + q * _CH, _CH)])
  rem = _ZROWS % _CH
  if rem:
    pltpu.sync_copy(zbuf.at[pl.ds(0, rem)],
                    accum.at[pl.ds(zbase + _ZROWS - rem, rem)])

  pltpu.sync_copy(cnt_h.at[pl.ds(wid * 16, 16)], cntb)
  cnt16 = cntb[pl.ds(0, 16)]
  nch = jnp.sum(jnp.where(lane == 0, cnt16, 0))
  nrounds = nch // _R

  plsc.subcore_barrier()

  def issue_idx(chunk, s):
    base = seg + chunk * _CH
    pltpu.async_copy(rows_h.at[pl.ds(base, _CH)], ibuf[s].at[pl.ds(0, _CH)],
                     isem[s])
    pltpu.async_copy(cols_h.at[pl.ds(base, _CH)], ibuf[s].at[pl.ds(_CH, _CH)],
                     isem[s])
    pltpu.async_copy(vals_h.at[pl.ds(base, _CH)],
                     ibuf[s].at[pl.ds(2 * _CH, _CH)], isem[s])

  def wait_idx(s):
    for f in range(3):
      pltpu.make_async_copy(rows_h.at[pl.ds(0, _CH)],
                            ibuf[s].at[pl.ds(f * _CH, _CH)], isem[s]).wait()

  def issue_gather(s):
    pltpu.async_copy(emb_h.at[ibuf[s].at[pl.ds(_CH, _CH)]], gath[s], gsem[s])

  def wait_gather(s):
    pltpu.make_async_copy(emb_h.at[ibuf[s].at[pl.ds(_CH, _CH)]], gath[s],
                          gsem[s]).wait()

  def issue_scatter(s):
    pltpu.async_copy(scl[s], accum.at[lidx[s]], ssem[s], add=True)

  def wait_scatter(s):
    pltpu.make_async_copy(scl[s], accum.at[lidx[s]], ssem[s]).wait()

  for k in range(_R):
    issue_idx(k, k)
  for k in range(_R - 1):
    wait_idx(k)
    issue_gather(k)

  def visit(c, s, first, last, tail_gather):
    wait_gather(s)
    if not first:
      wait_scatter(s)
    _scale_and_index(ibuf[s], gath[s], scl[s], lidx[s])
    if not last:
      issue_idx(c + _R, s)
    if (not last) or tail_gather:
      sp = (s + _R - 1) % _R
      wait_idx(sp)
      issue_gather(sp)
    issue_scatter(s)

  for s in range(_R):
    visit(s, s, True, False, False)

  def loop_body(j, carry):
    cb = _R * j
    for s in range(_R):
      visit(cb + s, s, False, False, False)
    return carry

  lax.fori_loop(1, nrounds - 1, loop_body, 0)

  cl = (nrounds - 1) * _R
  for s in range(_R):
    visit(cl + s, s, False, True, s == 0)

  for s in range(_R):
    wait_scatter(s)

  plsc.subcore_barrier()

  @pl.when(sid < _NS - 1)
  def _drain_main():
    pltpu.sync_copy(
        accum.at[pl.ds(sid * _DRAIN, _DRAIN)],
        out_h.at[pl.ds(cid * _ROWS_PER_CORE + sid * _DRAIN, _DRAIN)])

  @pl.when(sid == _NS - 1)
  def _drain_last():
    pltpu.sync_copy(
        accum.at[pl.ds((_NS - 1) * _DRAIN, _DRAIN_LAST)],
        out_h.at[pl.ds(cid * _ROWS_PER_CORE + (_NS - 1) * _DRAIN,
                       _DRAIN_LAST)])


_sc_layer = functools.partial(
    pl.kernel,
    out_type=jax.ShapeDtypeStruct((_N_NODES, _DIM), jnp.float32),
    mesh=plsc.VectorSubcoreMesh(
        core_axis_name="c", subcore_axis_name="s",
        num_cores=_NC, num_subcores=_NS),
    scratch_types=[
        pltpu.VMEM_SHARED((_ACC_ROWS, _DIM), jnp.float32),
        [pltpu.VMEM((_WORDS,), jnp.int32) for _ in range(_R)],
        [pltpu.VMEM((_CH, _DIM), jnp.float32) for _ in range(_R)],
        [pltpu.VMEM((_CH, _DIM), jnp.float32) for _ in range(_R)],
        [pltpu.VMEM((_CH,), jnp.int32) for _ in range(_R)],
        pltpu.VMEM((_CH, _DIM), jnp.float32),
        pltpu.VMEM((16,), jnp.int32),
        [pltpu.SemaphoreType.DMA for _ in range(_R)],
        [pltpu.SemaphoreType.DMA for _ in range(_R)],
        [pltpu.SemaphoreType.DMA for _ in range(_R)],
    ],
    compiler_params=pltpu.CompilerParams(
        needs_layout_passes=False, use_tc_tiling_on_sc=False),
)(_layer_body)


def _mean_body(au, ai, bu, bi, cu, ci, du, di, ou, oi):
  ou[...] = (au[...] + bu[...] + cu[...] + du[...]) * 0.25
  oi[...] = (ai[...] + bi[...] + ci[...] + di[...]) * 0.25


_MB = 25
_MROWS = _N_USERS // _MB
_spec_u = pl.BlockSpec((_MROWS, _DIM), lambda i: (i, 0))
_spec_i = pl.BlockSpec((_MROWS, _DIM), lambda i: (i + _MB, 0))

_mean4 = pl.pallas_call(
    _mean_body,
    grid=(_MB,),
    in_specs=[_spec_u, _spec_i] * 4,
    out_specs=[pl.BlockSpec((_MROWS, _DIM), lambda i: (i, 0))] * 2,
    out_shape=[jax.ShapeDtypeStruct((_N_USERS, _DIM), jnp.float32),
               jax.ShapeDtypeStruct((_N_ITEMS, _DIM), jnp.float32)],
)


def _pack_edges(adj_indices, adj_values):
  pad = _NCH * _CH - _N_EDGES
  rows = jnp.concatenate([adj_indices[0], jnp.zeros((pad,), jnp.int32)])
  cols = jnp.concatenate([adj_indices[1], jnp.zeros((pad,), jnp.int32)])
  vals = jnp.concatenate([adj_values, jnp.zeros((pad,), jnp.float32)])
  vbits = lax.bitcast_convert_type(vals, jnp.int32)
  packed = jnp.stack(
      [rows.reshape(_NCH, _CH), cols.reshape(_NCH, _CH),
       vbits.reshape(_NCH, _CH)], axis=1)
  return packed.reshape(-1)


def kernel(adj_indices, adj_values, user_emb, item_emb):
  packed = _pack_edges(adj_indices, adj_values)
  rows_l, cols_l, vals_l, cnts = _sc_partition(packed)
  emb0 = jnp.concatenate([user_emb, item_emb], axis=0)
  emb1 = _sc_layer(rows_l, cols_l, vals_l, cnts, emb0)
  emb2 = _sc_layer(rows_l, cols_l, vals_l, cnts, emb1)
  emb3 = _sc_layer(rows_l, cols_l, vals_l, cnts, emb2)
  user, item = _mean4(emb0, emb0, emb1, emb1, emb2, emb2, emb3, emb3)
  return (user, item)
